# baseline (speedup 1.0000x reference)
import functools

import jax
import jax.numpy as jnp
from jax import lax
from jax.experimental import pallas as pl
from jax.experimental.pallas import tpu as pltpu
from jax.experimental.pallas import tpu_sc as plsc

B, S, D = 4, 4096, 4096
L = 16
NCHUNK = 8
CHUNK = D // NCHUNK


def _pooling_call(hid_flat, mask_flat):
    mesh = plsc.VectorSubcoreMesh(core_axis_name="c", subcore_axis_name="s",
                                  num_cores=2, num_subcores=16)

    @functools.partial(
        pl.kernel,
        out_type=jax.ShapeDtypeStruct((B * D,), jnp.float32),
        mesh=mesh,
        scratch_types=[
            pltpu.VMEM((S,), jnp.int32),
            pltpu.VMEM((B * L,), jnp.int32),
            pltpu.VMEM((CHUNK,), jnp.float32),
        ],
    )
    def body(hid_hbm, mask_hbm, out_hbm, mrow_v, last_v, row_v):
        wid = lax.axis_index("s") * 2 + lax.axis_index("c")
        b = wid % B
        c = wid // B
        pltpu.sync_copy(mask_hbm.at[pl.ds(pl.multiple_of(b * S, S), S)], mrow_v)
        for b2 in range(B):
            pltpu.sync_copy(mask_hbm.at[pl.ds(b2 * S + S - L, L)],
                            last_v.at[pl.ds(b2 * L, L)])

        def step(i, acc):
            return acc + mrow_v[pl.ds(pl.multiple_of(i * L, L), L)]

        acc = lax.fori_loop(0, S // L, step, jnp.zeros((L,), jnp.int32))
        total = acc[0]
        for i in range(1, L):
            total = total + acc[i]
        lastsum = (last_v[pl.ds(0 * L, L)] + last_v[pl.ds(1 * L, L)] +
                   last_v[pl.ds(2 * L, L)] + last_v[pl.ds(3 * L, L)])
        lp = lastsum[L - 1]
        idx = jnp.where(lp == B, S - 1, (total - 1) & (S - 1))
        src = pl.multiple_of((b * S + idx) * D + c * CHUNK, CHUNK)
        pltpu.sync_copy(hid_hbm.at[pl.ds(src, CHUNK)], row_v)
        dst = pl.multiple_of(b * D + c * CHUNK, CHUNK)
        pltpu.sync_copy(row_v, out_hbm.at[pl.ds(dst, CHUNK)])

    return body(hid_flat, mask_flat)


def kernel(last_hidden_state, attention_mask):
    hid_flat = last_hidden_state.reshape(B * S * D)
    mask_flat = attention_mask.astype(jnp.int32).reshape(B * S)
    return _pooling_call(hid_flat, mask_flat).reshape(B, D)


# speedup vs baseline: 7.1371x; 7.1371x over previous
import functools

import jax
import jax.numpy as jnp
from jax import lax
from jax.experimental import pallas as pl
from jax.experimental.pallas import tpu as pltpu
from jax.experimental.pallas import tpu_sc as plsc

B, S, D = 4, 4096, 4096
L = 16


def _index_call(mask):
    mesh = plsc.VectorSubcoreMesh(core_axis_name="c", subcore_axis_name="s",
                                  num_cores=2, num_subcores=16)

    @functools.partial(
        pl.kernel,
        out_type=jax.ShapeDtypeStruct((L,), jnp.int32),
        mesh=mesh,
        scratch_types=[
            pltpu.VMEM((B, S), jnp.int32),
            pltpu.VMEM((L,), jnp.int32),
        ],
    )
    def body(mask_hbm, out_hbm, mask_v, idx_v):
        wid = lax.axis_index("s") * 2 + lax.axis_index("c")

        @pl.when(wid == 0)
        def _():
            pltpu.sync_copy(mask_hbm, mask_v)
            lanes = lax.iota(jnp.int32, L)
            out = jnp.zeros((L,), jnp.int32)
            lp = None
            totals = []
            for b in range(B):
                def step(i, acc, b=b):
                    return acc + mask_v[b, pl.ds(pl.multiple_of(i * L, L), L)]

                acc = lax.fori_loop(0, S // L, step, jnp.zeros((L,), jnp.int32))
                total = acc[0]
                for i in range(1, L):
                    total = total + acc[i]
                totals.append(total)
                last = mask_v[b, pl.ds(S - L, L)][L - 1]
                lp = last if lp is None else lp + last
            for b in range(B):
                idx = jnp.where(lp == B, S - 1, (totals[b] - 1) & (S - 1))
                out = jnp.where(lanes == b, idx, out)
            idx_v[...] = out
            pltpu.sync_copy(idx_v, out_hbm)

    return body(mask)


def _gather_body(idx_ref, hid_ref, out_ref):
    b = pl.program_id(0)
    r = idx_ref[b] % 8
    out_ref[...] = hid_ref[0, pl.ds(r, 1), :].reshape(1, 1, D)


def _gather_call(hid, idx16):
    grid_spec = pltpu.PrefetchScalarGridSpec(
        num_scalar_prefetch=1,
        grid=(B,),
        in_specs=[pl.BlockSpec((1, 8, D),
                               lambda b, idx_ref: (b, idx_ref[b] // 8, 0))],
        out_specs=pl.BlockSpec((1, 1, D), lambda b, idx_ref: (b, 0, 0)),
    )
    return pl.pallas_call(
        _gather_body,
        grid_spec=grid_spec,
        out_shape=jax.ShapeDtypeStruct((B, 1, D), jnp.float32),
    )(idx16, hid)


def kernel(last_hidden_state, attention_mask):
    idx16 = _index_call(attention_mask.astype(jnp.int32))
    return _gather_call(last_hidden_state, idx16).reshape(B, D)


# speedup vs baseline: 82.4501x; 11.5523x over previous
import jax
import jax.numpy as jnp
from jax.experimental import pallas as pl
from jax.experimental.pallas import tpu as pltpu

B, S, D = 4, 4096, 4096


def _pool_body(mask_ref, hid_ref, out_ref, sems):
    mask = mask_ref[...]
    totals = jnp.sum(mask, axis=1)
    lp = jnp.sum(mask_ref[:, pl.ds(S - 1, 1)])
    copies = []
    for b in range(B):
        idx = jnp.where(lp == B, S - 1, (totals[b] - 1) & (S - 1))
        copies.append(pltpu.make_async_copy(
            hid_ref.at[b, idx], out_ref.at[b], sems.at[b]))
    for c in copies:
        c.start()
    for c in copies:
        c.wait()


def kernel(last_hidden_state, attention_mask):
    return pl.pallas_call(
        _pool_body,
        in_specs=[
            pl.BlockSpec((B, S), lambda: (0, 0)),
            pl.BlockSpec(memory_space=pl.ANY),
        ],
        out_specs=pl.BlockSpec((B, D), lambda: (0, 0)),
        out_shape=jax.ShapeDtypeStruct((B, D), jnp.float32),
        scratch_shapes=[pltpu.SemaphoreType.DMA((B,))],
    )(attention_mask.astype(jnp.int32), last_hidden_state)
